# baseline (speedup 1.0000x reference)
import functools

import jax
import jax.numpy as jnp
import numpy as np
from jax.experimental import pallas as pl

B, L, D = 4, 8192, 768
TILE_L = 512
NL = L // TILE_L
L4 = L // 4
L16 = L // 16
TEMP_INV = 2.0


def _gumbel_host():
    cpu = jax.local_devices(backend="cpu")[0]
    with jax.default_device(cpu):
        gkey = jax.random.key(42)
        u = jax.random.uniform(gkey, (B, L, 2), minval=1e-7, maxval=1.0 - 1e-7)
        g = jnp.clip(-jnp.log(-jnp.log(u)), -6.0, 6.0)
        return np.asarray(jnp.transpose(g, (0, 2, 1)))


_GUMBEL_T = _gumbel_host()


def _router_pool_kernel(x_ref, g_ref, w1_ref, b1_ref, w2_ref, b2_ref, p4_ref,
                        xm4_ref, ps0_ref, ps1_ref):
    x = jnp.clip(x_ref[0], -4.0, 4.0)
    xb = x.astype(jnp.bfloat16)
    xm4_ref[0] = jnp.dot(p4_ref[...], xb,
                         preferred_element_type=jnp.float32).astype(jnp.bfloat16)
    h = jnp.dot(xb, w1_ref[...], preferred_element_type=jnp.float32) + b1_ref[0]
    h = jnp.maximum(h, 0.0).astype(jnp.bfloat16)
    logits = jnp.dot(h, w2_ref[...], preferred_element_type=jnp.float32) + b2_ref[0]
    lc = jnp.clip(logits, -15.0, 15.0)
    lt = lc.T
    z = (lt + g_ref[0]) * TEMP_INV
    d = z[0:1, :] - z[1:2, :]
    p0 = 1.0 / (1.0 + jnp.exp(-d))
    p0c = jnp.clip(p0, 1e-7, 1.0 - 1e-7)
    p1c = jnp.clip(1.0 - p0, 1e-7, 1.0 - 1e-7)
    s0 = p0c[:, 0:128]
    s1 = p1c[:, 0:128]
    for k in range(1, TILE_L // 128):
        s0 = s0 + p0c[:, k * 128:(k + 1) * 128]
        s1 = s1 + p1c[:, k * 128:(k + 1) * 128]
    ps0_ref[0, 0] = s0
    ps1_ref[0, 0] = s1


def _project_mix_kernel(xm4_ref, wp4_ref, bp4_ref, wp16_ref, bp16_ref,
                        mix_ref, p16_ref, out_ref):
    xm4 = xm4_ref[0]
    y4 = jnp.dot(xm4, wp4_ref[...], preferred_element_type=jnp.float32) + bp4_ref[0]
    y4 = jnp.clip(y4, -6.0, 6.0)
    xm16 = jnp.dot(p16_ref[...], xm4,
                   preferred_element_type=jnp.float32).astype(jnp.bfloat16)
    y16 = jnp.dot(xm16, wp16_ref[...],
                  preferred_element_type=jnp.float32) + bp16_ref[0]
    y16 = jnp.clip(y16, -6.0, 6.0)
    w4 = mix_ref[0, 0:1, 0:1]
    w16 = mix_ref[0, 0:1, 1:2]
    out_ref[0, :L16, :] = jnp.clip(w4 * y4[:L16] + w16 * y16, -6.0, 6.0)
    out_ref[0, L16:, :] = jnp.clip(w4 * y4[L16:], -6.0, 6.0)


@functools.partial(jax.jit, static_argnames=())
def kernel(visual_tokens, W1, b1, W2, b2, Wp4, bp4, Wp16, bp16):
    f32 = jnp.float32
    gt = jnp.asarray(_GUMBEL_T)

    w1t = W1.T.astype(jnp.bfloat16)
    w2t = W2.T.astype(jnp.bfloat16)
    b1r = b1.reshape(1, D).astype(f32)
    b2r = b2.reshape(1, 2).astype(f32)

    def _pool_matrix(rows, cols):
        sel = jnp.arange(rows)[:, None] == (jnp.arange(cols)[None, :] // 4)
        return jnp.where(sel, 0.25, 0.0).astype(jnp.bfloat16)

    p4 = _pool_matrix(TILE_L // 4, TILE_L)
    p16 = _pool_matrix(L16, L4)

    xm4, ps0, ps1 = pl.pallas_call(
        _router_pool_kernel,
        grid=(B, NL),
        in_specs=[
            pl.BlockSpec((1, TILE_L, D), lambda b, l: (b, l, 0)),
            pl.BlockSpec((1, 2, TILE_L), lambda b, l: (b, 0, l)),
            pl.BlockSpec((D, D), lambda b, l: (0, 0)),
            pl.BlockSpec((1, D), lambda b, l: (0, 0)),
            pl.BlockSpec((D, 2), lambda b, l: (0, 0)),
            pl.BlockSpec((1, 2), lambda b, l: (0, 0)),
            pl.BlockSpec((TILE_L // 4, TILE_L), lambda b, l: (0, 0)),
        ],
        out_specs=[
            pl.BlockSpec((1, TILE_L // 4, D), lambda b, l: (b, l, 0)),
            pl.BlockSpec((1, 1, 1, 128), lambda b, l: (b, l, 0, 0)),
            pl.BlockSpec((1, 1, 1, 128), lambda b, l: (b, l, 0, 0)),
        ],
        out_shape=[
            jax.ShapeDtypeStruct((B, L4, D), jnp.bfloat16),
            jax.ShapeDtypeStruct((B, NL, 1, 128), f32),
            jax.ShapeDtypeStruct((B, NL, 1, 128), f32),
        ],
    )(visual_tokens, gt, w1t, b1r, w2t, b2r, p4)

    m4 = ps0.sum(axis=(1, 2, 3)) / L
    m16 = ps1.sum(axis=(1, 2, 3)) / L
    wsum = m4 + m16 + 1e-7
    mix = jnp.stack([m4 / wsum, m16 / wsum], axis=-1)
    mix = mix.reshape(B, 1, 2).astype(f32)

    wp4t = Wp4.T.astype(jnp.bfloat16)
    wp16t = Wp16.T.astype(jnp.bfloat16)
    bp4r = bp4.reshape(1, D).astype(f32)
    bp16r = bp16.reshape(1, D).astype(f32)

    out = pl.pallas_call(
        _project_mix_kernel,
        grid=(B,),
        in_specs=[
            pl.BlockSpec((1, L4, D), lambda b: (b, 0, 0)),
            pl.BlockSpec((D, D), lambda b: (0, 0)),
            pl.BlockSpec((1, D), lambda b: (0, 0)),
            pl.BlockSpec((D, D), lambda b: (0, 0)),
            pl.BlockSpec((1, D), lambda b: (0, 0)),
            pl.BlockSpec((1, 1, 2), lambda b: (b, 0, 0)),
            pl.BlockSpec((L16, L4), lambda b: (0, 0)),
        ],
        out_specs=pl.BlockSpec((1, L4, D), lambda b: (b, 0, 0)),
        out_shape=jax.ShapeDtypeStruct((B, L4, D), f32),
    )(xm4, wp4t, bp4r, wp16t, bp16r, mix, p16)
    return out


# speedup vs baseline: 1.0839x; 1.0839x over previous
import functools

import jax
import jax.numpy as jnp
import numpy as np
from jax.experimental import pallas as pl

B, L, D = 4, 8192, 768
TILE_L = 1024
NL = L // TILE_L
L4 = L // 4
L16 = L // 16
TEMP_INV = 2.0


def _gumbel_host():
    cpu = jax.local_devices(backend="cpu")[0]
    with jax.default_device(cpu):
        gkey = jax.random.key(42)
        u = jax.random.uniform(gkey, (B, L, 2), minval=1e-7, maxval=1.0 - 1e-7)
        g = jnp.clip(-jnp.log(-jnp.log(u)), -6.0, 6.0)
        return np.asarray(jnp.transpose(g, (0, 2, 1)))


_GUMBEL_T = _gumbel_host()


def _router_pool_kernel(x_ref, g_ref, w1_ref, b1_ref, w2_ref, b2_ref, p4_ref,
                        xm4_ref, ps0_ref, ps1_ref):
    x = jnp.clip(x_ref[0], -4.0, 4.0)
    xb = x.astype(jnp.bfloat16)
    xm4_ref[0] = jnp.dot(p4_ref[...], xb,
                         preferred_element_type=jnp.float32).astype(jnp.bfloat16)
    h = jnp.dot(xb, w1_ref[...], preferred_element_type=jnp.float32) + b1_ref[0]
    h = jnp.maximum(h, 0.0).astype(jnp.bfloat16)
    logits = jnp.dot(h, w2_ref[...], preferred_element_type=jnp.float32) + b2_ref[0]
    lc = jnp.clip(logits, -15.0, 15.0)
    lt = lc.T
    z = (lt + g_ref[0]) * TEMP_INV
    d = z[0:1, :] - z[1:2, :]
    p0 = 1.0 / (1.0 + jnp.exp(-d))
    p0c = jnp.clip(p0, 1e-7, 1.0 - 1e-7)
    p1c = jnp.clip(1.0 - p0, 1e-7, 1.0 - 1e-7)
    s0 = p0c[:, 0:128]
    s1 = p1c[:, 0:128]
    for k in range(1, TILE_L // 128):
        s0 = s0 + p0c[:, k * 128:(k + 1) * 128]
        s1 = s1 + p1c[:, k * 128:(k + 1) * 128]
    ps0_ref[0, 0] = s0
    ps1_ref[0, 0] = s1


def _project_mix_kernel(xm4_ref, wp4_ref, bp4_ref, wp16_ref, bp16_ref,
                        mix_ref, p16_ref, out_ref):
    xm4 = xm4_ref[0]
    y4 = jnp.dot(xm4, wp4_ref[...], preferred_element_type=jnp.float32) + bp4_ref[0]
    y4 = jnp.clip(y4, -6.0, 6.0)
    xm16 = jnp.dot(p16_ref[...], xm4,
                   preferred_element_type=jnp.float32).astype(jnp.bfloat16)
    y16 = jnp.dot(xm16, wp16_ref[...],
                  preferred_element_type=jnp.float32) + bp16_ref[0]
    y16 = jnp.clip(y16, -6.0, 6.0)
    w4 = mix_ref[0, 0:1, 0:1]
    w16 = mix_ref[0, 0:1, 1:2]
    out_ref[0, :L16, :] = jnp.clip(w4 * y4[:L16] + w16 * y16, -6.0, 6.0)
    out_ref[0, L16:, :] = jnp.clip(w4 * y4[L16:], -6.0, 6.0)


@functools.partial(jax.jit, static_argnames=())
def kernel(visual_tokens, W1, b1, W2, b2, Wp4, bp4, Wp16, bp16):
    f32 = jnp.float32
    gt = jnp.asarray(_GUMBEL_T)

    w1t = W1.T.astype(jnp.bfloat16)
    w2t = W2.T.astype(jnp.bfloat16)
    b1r = b1.reshape(1, D).astype(f32)
    b2r = b2.reshape(1, 2).astype(f32)

    def _pool_matrix(rows, cols):
        sel = jnp.arange(rows)[:, None] == (jnp.arange(cols)[None, :] // 4)
        return jnp.where(sel, 0.25, 0.0).astype(jnp.bfloat16)

    p4 = _pool_matrix(TILE_L // 4, TILE_L)
    p16 = _pool_matrix(L16, L4)

    xm4, ps0, ps1 = pl.pallas_call(
        _router_pool_kernel,
        grid=(B, NL),
        in_specs=[
            pl.BlockSpec((1, TILE_L, D), lambda b, l: (b, l, 0)),
            pl.BlockSpec((1, 2, TILE_L), lambda b, l: (b, 0, l)),
            pl.BlockSpec((D, D), lambda b, l: (0, 0)),
            pl.BlockSpec((1, D), lambda b, l: (0, 0)),
            pl.BlockSpec((D, 2), lambda b, l: (0, 0)),
            pl.BlockSpec((1, 2), lambda b, l: (0, 0)),
            pl.BlockSpec((TILE_L // 4, TILE_L), lambda b, l: (0, 0)),
        ],
        out_specs=[
            pl.BlockSpec((1, TILE_L // 4, D), lambda b, l: (b, l, 0)),
            pl.BlockSpec((1, 1, 1, 128), lambda b, l: (b, l, 0, 0)),
            pl.BlockSpec((1, 1, 1, 128), lambda b, l: (b, l, 0, 0)),
        ],
        out_shape=[
            jax.ShapeDtypeStruct((B, L4, D), jnp.bfloat16),
            jax.ShapeDtypeStruct((B, NL, 1, 128), f32),
            jax.ShapeDtypeStruct((B, NL, 1, 128), f32),
        ],
    )(visual_tokens, gt, w1t, b1r, w2t, b2r, p4)

    m4 = ps0.sum(axis=(1, 2, 3)) / L
    m16 = ps1.sum(axis=(1, 2, 3)) / L
    wsum = m4 + m16 + 1e-7
    mix = jnp.stack([m4 / wsum, m16 / wsum], axis=-1)
    mix = mix.reshape(B, 1, 2).astype(f32)

    wp4t = Wp4.T.astype(jnp.bfloat16)
    wp16t = Wp16.T.astype(jnp.bfloat16)
    bp4r = bp4.reshape(1, D).astype(f32)
    bp16r = bp16.reshape(1, D).astype(f32)

    out = pl.pallas_call(
        _project_mix_kernel,
        grid=(B,),
        in_specs=[
            pl.BlockSpec((1, L4, D), lambda b: (b, 0, 0)),
            pl.BlockSpec((D, D), lambda b: (0, 0)),
            pl.BlockSpec((1, D), lambda b: (0, 0)),
            pl.BlockSpec((D, D), lambda b: (0, 0)),
            pl.BlockSpec((1, D), lambda b: (0, 0)),
            pl.BlockSpec((1, 1, 2), lambda b: (b, 0, 0)),
            pl.BlockSpec((L16, L4), lambda b: (0, 0)),
        ],
        out_specs=pl.BlockSpec((1, L4, D), lambda b: (b, 0, 0)),
        out_shape=jax.ShapeDtypeStruct((B, L4, D), f32),
    )(xm4, wp4t, bp4r, wp16t, bp16r, mix, p16)
    return out
